# baseline (speedup 1.0000x reference)
import functools

import jax
import jax.numpy as jnp
from jax import lax
from jax.experimental import pallas as pl
from jax.experimental.pallas import tpu as pltpu
from jax.experimental.pallas import tpu_sc as plsc


def _sc_gather(idx, table, n_chunks, chunk, nc, ns):
    nw = nc * ns
    rows_per_w = n_chunks * chunk
    n_total = nw * rows_per_w
    d = table.shape[1]

    mesh = plsc.VectorSubcoreMesh(core_axis_name="c", subcore_axis_name="s")
    nbuf = 4
    assert n_chunks % nbuf == 0 and n_chunks >= 2 * nbuf

    @functools.partial(
        pl.kernel,
        out_type=jax.ShapeDtypeStruct((n_total, d), jnp.float32),
        mesh=mesh,
        scratch_types=[
            pltpu.VMEM((n_chunks, chunk), jnp.int32),
            [pltpu.VMEM((chunk, d), jnp.float32)] * nbuf,
            [pltpu.SemaphoreType.DMA] * nbuf,
            [pltpu.SemaphoreType.DMA] * nbuf,
        ],
    )
    def gather_k(idx_hbm, table_hbm, out_hbm, idx_v, bufs, gs, os_):
        wid = lax.axis_index("s") * nc + lax.axis_index("c")
        base = wid * rows_per_w
        pltpu.sync_copy(idx_hbm.at[wid], idx_v)

        def g_copy(j, b):
            return pltpu.make_async_copy(table_hbm.at[idx_v.at[j]], bufs[b], gs[b])

        def o_copy(j, b):
            return pltpu.make_async_copy(
                bufs[b], out_hbm.at[pl.ds(base + j * chunk, chunk)], os_[b])

        g_copy(0, 0).start()
        g_copy(1, 1).start()
        for j in (0, 1):
            g_copy(j, j).wait()
            o_copy(j, j).start()
            g_copy(j + 2, j + 2).start()

        def body(p, carry):
            j0 = nbuf * p + 2
            for i in range(nbuf):
                j, b = j0 + i, (2 + i) % nbuf
                g_copy(j, b).wait()
                o_copy(j, b).start()
                o_copy(j - 2, (b + 2) % nbuf).wait()
                g_copy(j + 2, (b + 2) % nbuf).start()
            return carry

        lax.fori_loop(0, (n_chunks - 4) // nbuf, body, 0)
        for j in (n_chunks - 2, n_chunks - 1):
            b = j % nbuf
            g_copy(j, b).wait()
            o_copy(j, b).start()
            o_copy(j - 2, (b + 2) % nbuf).wait()
        for j in (n_chunks - 2, n_chunks - 1):
            o_copy(j, j % nbuf).wait()

    return gather_k(idx, table)


def kernel(indices, table):
    b0, b1 = indices.shape
    v, d = table.shape
    n = b0 * b1

    info = plsc.get_sparse_core_info()
    nc, ns = info.num_cores, info.num_subcores
    nw = nc * ns

    chunk = 32
    per_w = n // nw
    n_chunks = per_w // chunk
    assert n == nw * n_chunks * chunk, (n, nw, chunk)

    idx = indices.reshape(nw, n_chunks, chunk).astype(jnp.int32)
    out = _sc_gather(idx, table, n_chunks, chunk, nc, ns)
    return out.reshape(b0, b1, d)


# speedup vs baseline: 3.1898x; 3.1898x over previous
import functools

import jax
import jax.numpy as jnp
from jax import lax
from jax.experimental import pallas as pl
from jax.experimental.pallas import tpu as pltpu
from jax.experimental.pallas import tpu_sc as plsc


def _sc_gather(idx, table, n_chunks, chunk, nc, ns):
    nw = nc * ns
    rows_per_w = n_chunks * chunk
    n_total = nw * rows_per_w
    d = table.shape[1]

    mesh = plsc.VectorSubcoreMesh(core_axis_name="c", subcore_axis_name="s")
    nbuf = 4
    assert n_chunks % nbuf == 0 and n_chunks >= 2 * nbuf

    @functools.partial(
        pl.kernel,
        out_type=jax.ShapeDtypeStruct((n_total, d), jnp.float32),
        mesh=mesh,
        scratch_types=[
            pltpu.VMEM((n_chunks, chunk), jnp.int32),
            [pltpu.VMEM((chunk, d), jnp.float32)] * nbuf,
            [pltpu.SemaphoreType.DMA] * nbuf,
            [pltpu.SemaphoreType.DMA] * nbuf,
        ],
    )
    def gather_k(idx_hbm, table_hbm, out_hbm, idx_v, bufs, gs, os_):
        wid = lax.axis_index("s") * nc + lax.axis_index("c")
        base = wid * rows_per_w
        pltpu.sync_copy(idx_hbm.at[wid], idx_v)

        def g_copy(j, b):
            return pltpu.make_async_copy(table_hbm.at[idx_v.at[j]], bufs[b], gs[b])

        def o_copy(j, b):
            return pltpu.make_async_copy(
                bufs[b], out_hbm.at[pl.ds(base + j * chunk, chunk)], os_[b])

        g_copy(0, 0).start()
        g_copy(1, 1).start()
        for j in (0, 1):
            g_copy(j, j).wait()
            o_copy(j, j).start()
            g_copy(j + 2, j + 2).start()

        def body(p, carry):
            j0 = nbuf * p + 2
            for i in range(nbuf):
                j, b = j0 + i, (2 + i) % nbuf
                g_copy(j, b).wait()
                o_copy(j, b).start()
                o_copy(j - 2, (b + 2) % nbuf).wait()
                g_copy(j + 2, (b + 2) % nbuf).start()
            return carry

        lax.fori_loop(0, (n_chunks - 4) // nbuf, body, 0)
        for j in (n_chunks - 2, n_chunks - 1):
            b = j % nbuf
            g_copy(j, b).wait()
            o_copy(j, b).start()
            o_copy(j - 2, (b + 2) % nbuf).wait()
        for j in (n_chunks - 2, n_chunks - 1):
            o_copy(j, j % nbuf).wait()

    return gather_k(idx, table)


def kernel(indices, table):
    b0, b1 = indices.shape
    v, d = table.shape
    n = b0 * b1

    info = plsc.get_sparse_core_info()
    nc, ns = info.num_cores, info.num_subcores
    nw = nc * ns

    chunk = 32
    per_w = n // nw
    n_chunks = per_w // chunk
    assert n == nw * n_chunks * chunk, (n, nw, chunk)

    idx = indices.T.reshape(nw, n_chunks, chunk).astype(jnp.int32)
    out = _sc_gather(idx, table, n_chunks, chunk, nc, ns)
    return out.reshape(b1, b0, d).transpose(1, 0, 2)
